# baseline (speedup 1.0000x reference)
import functools

import jax
import jax.numpy as jnp
from jax import lax
from jax.experimental import pallas as pl
from jax.experimental.pallas import tpu as pltpu
from jax.experimental.pallas import tpu_sc as plsc

MU = 0.1
EPS = 1e-08

NC = 2
NS = 16
NW = NC * NS
LANES = 16

NCHUNK = 4
TOTAL_CHUNKS = NW * NCHUNK
UNROLL = 8


def _sc_gather_ratio(n, t, chunk, t_out):
    mesh = plsc.VectorSubcoreMesh(
        core_axis_name="c", subcore_axis_name="s",
        num_cores=NC, num_subcores=NS)

    @functools.partial(
        pl.kernel,
        out_type=jax.ShapeDtypeStruct((t_out,), jnp.float32),
        mesh=mesh,
        scratch_types=[
            [pltpu.VMEM((3 * chunk,), jnp.int32) for _ in range(2)],
            [pltpu.VMEM((chunk,), jnp.int32) for _ in range(2)],
            [pltpu.VMEM((chunk,), jnp.int32) for _ in range(2)],
            [pltpu.VMEM((chunk,), jnp.float32) for _ in range(2)],
            [pltpu.VMEM((chunk,), jnp.float32) for _ in range(2)],
            pltpu.VMEM((chunk,), jnp.float32),
            pltpu.VMEM((n,), jnp.int32),
            pltpu.VMEM((n,), jnp.float32),
            [pltpu.SemaphoreType.DMA for _ in range(2)],
        ],
        compiler_params=pltpu.CompilerParams(needs_layout_passes=False),
    )
    def sc_kernel(kflat, trip, out,
                  trip_v, idx_il, idx_ij, val_il, val_ij,
                  ratio_v, didx_v, diag_v, sems):
        wid = lax.axis_index("s") * NC + lax.axis_index("c")

        @plsc.parallel_loop(0, n, step=LANES, unroll=UNROLL)
        def _(g):
            didx_v[pl.ds(g, LANES)] = (lax.iota(jnp.int32, LANES) + g) * (n + 1)
        pltpu.async_copy(kflat.at[didx_v], diag_v, sems[0]).wait()

        def chunk_base(c):
            cid = wid * NCHUNK + c
            return jnp.minimum(cid * chunk, t - chunk)

        def stage_fire(c):
            b = c % 2
            base = chunk_base(c)
            pltpu.sync_copy(trip.at[pl.ds(3 * base, 3 * chunk)], trip_v[b])
            tv, il, ij = trip_v[b], idx_il[b], idx_ij[b]

            @plsc.parallel_loop(0, chunk, step=LANES, unroll=UNROLL)
            def _(g):
                r3 = (lax.iota(jnp.int32, LANES) + g) * 3
                iv = plsc.load_gather(tv, [r3])
                jv = plsc.load_gather(tv, [r3 + 1])
                lv = plsc.load_gather(tv, [r3 + 2])
                row = iv * n
                il[pl.ds(g, LANES)] = row + lv
                ij[pl.ds(g, LANES)] = row + jv

            cps = (pltpu.async_copy(kflat.at[il], val_il[b], sems[b]),
                   pltpu.async_copy(kflat.at[ij], val_ij[b], sems[b]))
            return cps

        def drain_compute(c, cps):
            b = c % 2
            cps[0].wait()
            cps[1].wait()
            tv, vil_v, vij_v = trip_v[b], val_il[b], val_ij[b]

            @plsc.parallel_loop(0, chunk, step=LANES, unroll=UNROLL)
            def _(g):
                r3 = (lax.iota(jnp.int32, LANES) + g) * 3
                iv = plsc.load_gather(tv, [r3])
                jv = plsc.load_gather(tv, [r3 + 1])
                lv = plsc.load_gather(tv, [r3 + 2])
                di = plsc.load_gather(diag_v, [iv])
                dj = plsc.load_gather(diag_v, [jv])
                dl = plsc.load_gather(diag_v, [lv])
                vil = vil_v[pl.ds(g, LANES)]
                vij = vij_v[pl.ds(g, LANES)]
                d_il = (-2.0 * vil + di) + dl
                d_ij = (-2.0 * vij + di) + dj
                numer = jnp.maximum(MU + d_il, EPS)
                denom = jnp.maximum((2.0 * MU + d_ij) + d_il, EPS)
                ratio_v[pl.ds(g, LANES)] = denom / numer

            pltpu.sync_copy(ratio_v, out.at[pl.ds(chunk_base(c), chunk)])

        inflight = stage_fire(0)
        for c in range(NCHUNK):
            nxt = stage_fire(c + 1) if c + 1 < NCHUNK else None
            drain_compute(c, inflight)
            inflight = nxt

    return sc_kernel


def _tc_log_sum(t, rows, cols):
    def body(x_ref, o_ref):
        x = x_ref[...]
        r = lax.broadcasted_iota(jnp.int32, (rows, cols), 0)
        c = lax.broadcasted_iota(jnp.int32, (rows, cols), 1)
        flat = r * cols + c
        val = jnp.where(flat < t, jnp.log(x), 0.0)
        o_ref[0, 0] = jnp.sum(val)

    return pl.pallas_call(
        body,
        out_specs=pl.BlockSpec(memory_space=pltpu.SMEM),
        out_shape=jax.ShapeDtypeStruct((1, 1), jnp.float32),
    )


def kernel(k, triplets):
    n = k.shape[0]
    t = triplets.shape[0]
    chunk = -(-t // TOTAL_CHUNKS)
    chunk = -(-chunk // LANES) * LANES
    cols = 1024
    t_out = TOTAL_CHUNKS * chunk
    rows = -(-(-(-t_out // cols)) // 8) * 8
    t_out = rows * cols

    kflat = k.reshape(-1)
    tripflat = triplets.reshape(-1)

    ratio = _sc_gather_ratio(n, t, chunk, t_out)(kflat, tripflat)
    total = _tc_log_sum(t, rows, cols)(ratio.reshape(rows, cols))
    return total[0, 0]


# speedup vs baseline: 18.1456x; 18.1456x over previous
import functools

import jax
import jax.numpy as jnp
from jax import lax
from jax.experimental import pallas as pl
from jax.experimental.pallas import tpu as pltpu
from jax.experimental.pallas import tpu_sc as plsc

MU = 0.1
EPS = 1e-08

NC = 2
NS = 16
NW = NC * NS
LANES = 16

NCHUNK = 4
TOTAL_CHUNKS = NW * NCHUNK
UNROLL = 8


def _sc_gather_ratio(n, t, chunk, t_out):
    mesh = plsc.VectorSubcoreMesh(
        core_axis_name="c", subcore_axis_name="s",
        num_cores=NC, num_subcores=NS)

    @functools.partial(
        pl.kernel,
        out_type=jax.ShapeDtypeStruct((t_out,), jnp.float32),
        mesh=mesh,
        scratch_types=[
            [pltpu.VMEM((chunk,), jnp.int32) for _ in range(2)],
            [pltpu.VMEM((chunk,), jnp.int32) for _ in range(2)],
            [pltpu.VMEM((chunk,), jnp.int32) for _ in range(2)],
            [pltpu.VMEM((chunk,), jnp.int32) for _ in range(2)],
            [pltpu.VMEM((chunk,), jnp.int32) for _ in range(2)],
            [pltpu.VMEM((chunk,), jnp.float32) for _ in range(2)],
            [pltpu.VMEM((chunk,), jnp.float32) for _ in range(2)],
            pltpu.VMEM((chunk,), jnp.float32),
            pltpu.VMEM((n,), jnp.int32),
            pltpu.VMEM((n,), jnp.float32),
            [pltpu.SemaphoreType.DMA for _ in range(2)],
        ],
        compiler_params=pltpu.CompilerParams(needs_layout_passes=False),
    )
    def sc_kernel(kflat, ti, tj, tl, out,
                  iv_b, jv_b, lv_b, idx_il, idx_ij, val_il, val_ij,
                  ratio_v, didx_v, diag_v, sems):
        wid = lax.axis_index("s") * NC + lax.axis_index("c")

        @plsc.parallel_loop(0, n, step=LANES, unroll=UNROLL)
        def _(g):
            didx_v[pl.ds(g, LANES)] = (lax.iota(jnp.int32, LANES) + g) * (n + 1)
        pltpu.async_copy(kflat.at[didx_v], diag_v, sems[0]).wait()

        def chunk_base(c):
            cid = wid * NCHUNK + c
            return jnp.minimum(cid * chunk, t - chunk)

        def stage_fire(c):
            b = c % 2
            base = chunk_base(c)
            pltpu.sync_copy(ti.at[pl.ds(base, chunk)], iv_b[b])
            pltpu.sync_copy(tj.at[pl.ds(base, chunk)], jv_b[b])
            pltpu.sync_copy(tl.at[pl.ds(base, chunk)], lv_b[b])
            iv_v, jv_v, lv_v = iv_b[b], jv_b[b], lv_b[b]
            il, ij = idx_il[b], idx_ij[b]

            @plsc.parallel_loop(0, chunk, step=LANES, unroll=UNROLL)
            def _(g):
                iv = iv_v[pl.ds(g, LANES)]
                row = iv * n
                il[pl.ds(g, LANES)] = row + lv_v[pl.ds(g, LANES)]
                ij[pl.ds(g, LANES)] = row + jv_v[pl.ds(g, LANES)]

            cps = (pltpu.async_copy(kflat.at[il], val_il[b], sems[b]),
                   pltpu.async_copy(kflat.at[ij], val_ij[b], sems[b]))
            return cps

        def drain_compute(c, cps):
            b = c % 2
            cps[0].wait()
            cps[1].wait()
            iv_v, jv_v, lv_v = iv_b[b], jv_b[b], lv_b[b]
            vil_v, vij_v = val_il[b], val_ij[b]

            @plsc.parallel_loop(0, chunk, step=LANES, unroll=UNROLL)
            def _(g):
                iv = iv_v[pl.ds(g, LANES)]
                jv = jv_v[pl.ds(g, LANES)]
                lv = lv_v[pl.ds(g, LANES)]
                di = plsc.load_gather(diag_v, [iv])
                dj = plsc.load_gather(diag_v, [jv])
                dl = plsc.load_gather(diag_v, [lv])
                vil = vil_v[pl.ds(g, LANES)]
                vij = vij_v[pl.ds(g, LANES)]
                d_il = (-2.0 * vil + di) + dl
                d_ij = (-2.0 * vij + di) + dj
                numer = jnp.maximum(MU + d_il, EPS)
                denom = jnp.maximum((2.0 * MU + d_ij) + d_il, EPS)
                ratio_v[pl.ds(g, LANES)] = denom / numer

            pltpu.sync_copy(ratio_v, out.at[pl.ds(chunk_base(c), chunk)])

        inflight = stage_fire(0)
        for c in range(NCHUNK):
            nxt = stage_fire(c + 1) if c + 1 < NCHUNK else None
            drain_compute(c, inflight)
            inflight = nxt

    return sc_kernel


def _tc_log_sum(t, rows, cols):
    def body(x_ref, o_ref):
        x = x_ref[...]
        r = lax.broadcasted_iota(jnp.int32, (rows, cols), 0)
        c = lax.broadcasted_iota(jnp.int32, (rows, cols), 1)
        flat = r * cols + c
        val = jnp.where(flat < t, jnp.log(x), 0.0)
        o_ref[0, 0] = jnp.sum(val)

    return pl.pallas_call(
        body,
        out_specs=pl.BlockSpec(memory_space=pltpu.SMEM),
        out_shape=jax.ShapeDtypeStruct((1, 1), jnp.float32),
    )


def kernel(k, triplets):
    n = k.shape[0]
    t = triplets.shape[0]
    chunk = -(-t // TOTAL_CHUNKS)
    chunk = -(-chunk // LANES) * LANES
    cols = 1024
    t_out = TOTAL_CHUNKS * chunk
    rows = -(-(-(-t_out // cols)) // 8) * 8
    t_out = rows * cols

    kflat = k.reshape(-1)
    ti = triplets[:, 0]
    tj = triplets[:, 1]
    tl = triplets[:, 2]

    ratio = _sc_gather_ratio(n, t, chunk, t_out)(kflat, ti, tj, tl)
    total = _tc_log_sum(t, rows, cols)(ratio.reshape(rows, cols))
    return total[0, 0]


# speedup vs baseline: 18.5692x; 1.0233x over previous
import functools

import jax
import jax.numpy as jnp
from jax import lax
from jax.experimental import pallas as pl
from jax.experimental.pallas import tpu as pltpu
from jax.experimental.pallas import tpu_sc as plsc

MU = 0.1
EPS = 1e-08
LN2 = 0.6931471805599453

NC = 2
NS = 16
NW = NC * NS
LANES = 16

NCHUNK = 4
TOTAL_CHUNKS = NW * NCHUNK
UNROLL = 8


def _vln(x):
    bits = plsc.bitcast(x, jnp.int32)
    e = (bits >> 23) - 127
    m = plsc.bitcast((bits & 0x7FFFFF) | 0x3F800000, jnp.float32)
    mgt = m >= 1.5
    m2 = jnp.where(mgt, m * 0.5, m)
    ef = (e + mgt.astype(jnp.int32)).astype(jnp.float32)
    z = m2 - 1.0
    s = z / (z + 2.0)
    s2 = s * s
    h = 1.0 / 7.0
    h = h * s2 + 1.0 / 5.0
    h = h * s2 + 1.0 / 3.0
    h = h * s2 + 1.0
    return ef * LN2 + 2.0 * s * h


def _sc_loss_partials(n, t, chunk):
    mesh = plsc.VectorSubcoreMesh(
        core_axis_name="c", subcore_axis_name="s",
        num_cores=NC, num_subcores=NS)

    @functools.partial(
        pl.kernel,
        out_type=jax.ShapeDtypeStruct((NW * LANES,), jnp.float32),
        mesh=mesh,
        scratch_types=[
            [pltpu.VMEM((chunk,), jnp.int32) for _ in range(2)],
            [pltpu.VMEM((chunk,), jnp.int32) for _ in range(2)],
            [pltpu.VMEM((chunk,), jnp.int32) for _ in range(2)],
            [pltpu.VMEM((chunk,), jnp.int32) for _ in range(2)],
            [pltpu.VMEM((chunk,), jnp.int32) for _ in range(2)],
            [pltpu.VMEM((chunk,), jnp.float32) for _ in range(2)],
            [pltpu.VMEM((chunk,), jnp.float32) for _ in range(2)],
            pltpu.VMEM((LANES,), jnp.float32),
            pltpu.VMEM((n,), jnp.int32),
            pltpu.VMEM((n,), jnp.float32),
            [pltpu.SemaphoreType.DMA for _ in range(2)],
        ],
        compiler_params=pltpu.CompilerParams(needs_layout_passes=False),
    )
    def sc_kernel(kflat, ti, tj, tl, out,
                  iv_b, jv_b, lv_b, idx_il, idx_ij, val_il, val_ij,
                  part_v, didx_v, diag_v, sems):
        wid = lax.axis_index("s") * NC + lax.axis_index("c")

        @plsc.parallel_loop(0, n, step=LANES, unroll=UNROLL)
        def _(g):
            didx_v[pl.ds(g, LANES)] = (lax.iota(jnp.int32, LANES) + g) * (n + 1)
        pltpu.async_copy(kflat.at[didx_v], diag_v, sems[0]).wait()

        def chunk_base(c):
            cid = wid * NCHUNK + c
            return jnp.minimum(cid * chunk, t - chunk)

        def stage_fire(c):
            b = c % 2
            base = chunk_base(c)
            pltpu.sync_copy(ti.at[pl.ds(base, chunk)], iv_b[b])
            pltpu.sync_copy(tj.at[pl.ds(base, chunk)], jv_b[b])
            pltpu.sync_copy(tl.at[pl.ds(base, chunk)], lv_b[b])
            iv_v, jv_v, lv_v = iv_b[b], jv_b[b], lv_b[b]
            il, ij = idx_il[b], idx_ij[b]

            @plsc.parallel_loop(0, chunk, step=LANES, unroll=UNROLL)
            def _(g):
                iv = iv_v[pl.ds(g, LANES)]
                row = iv * n
                il[pl.ds(g, LANES)] = row + lv_v[pl.ds(g, LANES)]
                ij[pl.ds(g, LANES)] = row + jv_v[pl.ds(g, LANES)]

            cps = (pltpu.async_copy(kflat.at[il], val_il[b], sems[b]),
                   pltpu.async_copy(kflat.at[ij], val_ij[b], sems[b]))
            return cps

        def drain_compute(c, cps, acc):
            b = c % 2
            cps[0].wait()
            cps[1].wait()
            iv_v, jv_v, lv_v = iv_b[b], jv_b[b], lv_b[b]
            vil_v, vij_v = val_il[b], val_ij[b]
            cid = wid * NCHUNK + c
            thresh = cid * chunk - chunk_base(c)

            @plsc.parallel_loop(0, chunk, step=LANES, unroll=UNROLL,
                                carry=acc)
            def acc_loop(g, a):
                iv = iv_v[pl.ds(g, LANES)]
                jv = jv_v[pl.ds(g, LANES)]
                lv = lv_v[pl.ds(g, LANES)]
                di = plsc.load_gather(diag_v, [iv])
                dj = plsc.load_gather(diag_v, [jv])
                dl = plsc.load_gather(diag_v, [lv])
                vil = vil_v[pl.ds(g, LANES)]
                vij = vij_v[pl.ds(g, LANES)]
                d_il = (-2.0 * vil + di) + dl
                d_ij = (-2.0 * vij + di) + dj
                numer = jnp.maximum(MU + d_il, EPS)
                denom = jnp.maximum((2.0 * MU + d_ij) + d_il, EPS)
                term = _vln(denom / numer)
                keep = (lax.iota(jnp.int32, LANES) + g) >= thresh
                return a + jnp.where(keep, term, 0.0)

            return acc_loop

        acc = jnp.zeros((LANES,), jnp.float32)
        inflight = stage_fire(0)
        for c in range(NCHUNK):
            nxt = stage_fire(c + 1) if c + 1 < NCHUNK else None
            acc = drain_compute(c, inflight, acc)
            inflight = nxt

        part_v[...] = acc
        pltpu.sync_copy(part_v, out.at[pl.ds(wid * LANES, LANES)])

    return sc_kernel


def kernel(k, triplets):
    n = k.shape[0]
    t = triplets.shape[0]
    chunk = -(-t // TOTAL_CHUNKS)
    chunk = -(-chunk // LANES) * LANES

    kflat = k.reshape(-1)
    ti = triplets[:, 0]
    tj = triplets[:, 1]
    tl = triplets[:, 2]

    partials = _sc_loss_partials(n, t, chunk)(kflat, ti, tj, tl)
    return jnp.sum(partials)


# speedup vs baseline: 18.7967x; 1.0122x over previous
import functools

import jax
import jax.numpy as jnp
from jax import lax
from jax.experimental import pallas as pl
from jax.experimental.pallas import tpu as pltpu
from jax.experimental.pallas import tpu_sc as plsc

MU = 0.1
EPS = 1e-08
LN2 = 0.6931471805599453

NC = 2
NS = 16
NW = NC * NS
LANES = 16

NCHUNK = 8
DEPTH = 3
TOTAL_CHUNKS = NW * NCHUNK
UNROLL = 8


def _vln(x):
    bits = plsc.bitcast(x, jnp.int32)
    e = (bits >> 23) - 127
    m = plsc.bitcast((bits & 0x7FFFFF) | 0x3F800000, jnp.float32)
    mgt = m >= 1.5
    m2 = jnp.where(mgt, m * 0.5, m)
    ef = (e + mgt.astype(jnp.int32)).astype(jnp.float32)
    z = m2 - 1.0
    s = z / (z + 2.0)
    s2 = s * s
    h = 1.0 / 7.0
    h = h * s2 + 1.0 / 5.0
    h = h * s2 + 1.0 / 3.0
    h = h * s2 + 1.0
    return ef * LN2 + 2.0 * s * h


def _sc_loss_partials(n, t, chunk):
    mesh = plsc.VectorSubcoreMesh(
        core_axis_name="c", subcore_axis_name="s",
        num_cores=NC, num_subcores=NS)

    @functools.partial(
        pl.kernel,
        out_type=jax.ShapeDtypeStruct((NW * LANES,), jnp.float32),
        mesh=mesh,
        scratch_types=[
            [pltpu.VMEM((chunk,), jnp.int32) for _ in range(DEPTH)],
            [pltpu.VMEM((chunk,), jnp.int32) for _ in range(DEPTH)],
            [pltpu.VMEM((chunk,), jnp.int32) for _ in range(DEPTH)],
            [pltpu.VMEM((chunk,), jnp.int32) for _ in range(DEPTH)],
            [pltpu.VMEM((chunk,), jnp.int32) for _ in range(DEPTH)],
            [pltpu.VMEM((chunk,), jnp.float32) for _ in range(DEPTH)],
            [pltpu.VMEM((chunk,), jnp.float32) for _ in range(DEPTH)],
            pltpu.VMEM((LANES,), jnp.float32),
            pltpu.VMEM((n,), jnp.int32),
            pltpu.VMEM((n,), jnp.float32),
            [pltpu.SemaphoreType.DMA for _ in range(DEPTH)],
        ],
        compiler_params=pltpu.CompilerParams(needs_layout_passes=False),
    )
    def sc_kernel(kflat, ti, tj, tl, out,
                  iv_b, jv_b, lv_b, idx_il, idx_ij, val_il, val_ij,
                  part_v, didx_v, diag_v, sems):
        wid = lax.axis_index("s") * NC + lax.axis_index("c")

        @plsc.parallel_loop(0, n, step=LANES, unroll=UNROLL)
        def _(g):
            didx_v[pl.ds(g, LANES)] = (lax.iota(jnp.int32, LANES) + g) * (n + 1)
        pltpu.async_copy(kflat.at[didx_v], diag_v, sems[0]).wait()

        def chunk_base(c):
            cid = wid * NCHUNK + c
            return jnp.minimum(cid * chunk, t - chunk)

        def stage_fire(c):
            b = c % DEPTH
            base = chunk_base(c)
            pltpu.sync_copy(ti.at[pl.ds(base, chunk)], iv_b[b])
            pltpu.sync_copy(tj.at[pl.ds(base, chunk)], jv_b[b])
            pltpu.sync_copy(tl.at[pl.ds(base, chunk)], lv_b[b])
            iv_v, jv_v, lv_v = iv_b[b], jv_b[b], lv_b[b]
            il, ij = idx_il[b], idx_ij[b]

            @plsc.parallel_loop(0, chunk, step=LANES, unroll=UNROLL)
            def _(g):
                iv = iv_v[pl.ds(g, LANES)]
                row = iv * n
                il[pl.ds(g, LANES)] = row + lv_v[pl.ds(g, LANES)]
                ij[pl.ds(g, LANES)] = row + jv_v[pl.ds(g, LANES)]

            cps = (pltpu.async_copy(kflat.at[il], val_il[b], sems[b]),
                   pltpu.async_copy(kflat.at[ij], val_ij[b], sems[b]))
            return cps

        def drain_compute(c, cps, acc):
            b = c % DEPTH
            cps[0].wait()
            cps[1].wait()
            iv_v, jv_v, lv_v = iv_b[b], jv_b[b], lv_b[b]
            vil_v, vij_v = val_il[b], val_ij[b]
            cid = wid * NCHUNK + c
            thresh = cid * chunk - chunk_base(c)

            @plsc.parallel_loop(0, chunk, step=LANES, unroll=UNROLL,
                                carry=acc)
            def acc_loop(g, a):
                iv = iv_v[pl.ds(g, LANES)]
                jv = jv_v[pl.ds(g, LANES)]
                lv = lv_v[pl.ds(g, LANES)]
                di = plsc.load_gather(diag_v, [iv])
                dj = plsc.load_gather(diag_v, [jv])
                dl = plsc.load_gather(diag_v, [lv])
                vil = vil_v[pl.ds(g, LANES)]
                vij = vij_v[pl.ds(g, LANES)]
                d_il = (-2.0 * vil + di) + dl
                d_ij = (-2.0 * vij + di) + dj
                numer = jnp.maximum(MU + d_il, EPS)
                denom = jnp.maximum((2.0 * MU + d_ij) + d_il, EPS)
                term = _vln(denom / numer)
                keep = (lax.iota(jnp.int32, LANES) + g) >= thresh
                return a + jnp.where(keep, term, 0.0)

            return acc_loop

        acc = jnp.zeros((LANES,), jnp.float32)
        inflight = [stage_fire(c) for c in range(DEPTH - 1)]
        for c in range(NCHUNK):
            if c + DEPTH - 1 < NCHUNK:
                inflight.append(stage_fire(c + DEPTH - 1))
            acc = drain_compute(c, inflight.pop(0), acc)

        part_v[...] = acc
        pltpu.sync_copy(part_v, out.at[pl.ds(wid * LANES, LANES)])

    return sc_kernel


def kernel(k, triplets):
    n = k.shape[0]
    t = triplets.shape[0]
    chunk = -(-t // TOTAL_CHUNKS)
    chunk = -(-chunk // LANES) * LANES

    kflat = k.reshape(-1)
    ti = triplets[:, 0]
    tj = triplets[:, 1]
    tl = triplets[:, 2]

    partials = _sc_loss_partials(n, t, chunk)(kflat, ti, tj, tl)
    return jnp.sum(partials)


# speedup vs baseline: 19.0923x; 1.0157x over previous
import functools

import jax
import jax.numpy as jnp
from jax import lax
from jax.experimental import pallas as pl
from jax.experimental.pallas import tpu as pltpu
from jax.experimental.pallas import tpu_sc as plsc

MU = 0.1
EPS = 1e-08
LN2 = 0.6931471805599453

NC = 2
NS = 16
NW = NC * NS
LANES = 16

NCHUNK = 8
DEPTH = 3
TOTAL_CHUNKS = NW * NCHUNK
UNROLL = 8


def _vln(x):
    bits = plsc.bitcast(x, jnp.int32)
    e = (bits >> 23) - 127
    m = plsc.bitcast((bits & 0x7FFFFF) | 0x3F800000, jnp.float32)
    mgt = m >= 1.5
    m2 = jnp.where(mgt, m * 0.5, m)
    ef = (e + mgt.astype(jnp.int32)).astype(jnp.float32)
    z = m2 - 1.0
    s = z / (z + 2.0)
    s2 = s * s
    h = 1.0 / 7.0
    h = h * s2 + 1.0 / 5.0
    h = h * s2 + 1.0 / 3.0
    h = h * s2 + 1.0
    return ef * LN2 + 2.0 * s * h


def _sc_loss_partials(n, t, chunk):
    mesh = plsc.VectorSubcoreMesh(
        core_axis_name="c", subcore_axis_name="s",
        num_cores=NC, num_subcores=NS)

    @functools.partial(
        pl.kernel,
        out_type=jax.ShapeDtypeStruct((NW * LANES,), jnp.float32),
        mesh=mesh,
        scratch_types=[
            [pltpu.VMEM((chunk,), jnp.int32) for _ in range(DEPTH)],
            [pltpu.VMEM((chunk,), jnp.int32) for _ in range(DEPTH)],
            [pltpu.VMEM((chunk,), jnp.int32) for _ in range(DEPTH)],
            [pltpu.VMEM((chunk,), jnp.int32) for _ in range(DEPTH)],
            [pltpu.VMEM((chunk,), jnp.int32) for _ in range(DEPTH)],
            [pltpu.VMEM((chunk,), jnp.float32) for _ in range(DEPTH)],
            [pltpu.VMEM((chunk,), jnp.float32) for _ in range(DEPTH)],
            pltpu.VMEM((LANES,), jnp.float32),
            pltpu.VMEM((n,), jnp.int32),
            pltpu.VMEM((n,), jnp.float32),
            [pltpu.SemaphoreType.DMA for _ in range(DEPTH)],
            pltpu.SemaphoreType.DMA,
            pltpu.SemaphoreType.DMA,
        ],
        compiler_params=pltpu.CompilerParams(needs_layout_passes=False),
    )
    def sc_kernel(kflat, ti, tj, tl, out,
                  iv_b, jv_b, lv_b, idx_il, idx_ij, val_il, val_ij,
                  part_v, didx_v, diag_v, sems, ssem, dsem):
        wid = lax.axis_index("s") * NC + lax.axis_index("c")

        @plsc.parallel_loop(0, n, step=LANES, unroll=UNROLL)
        def _(g):
            didx_v[pl.ds(g, LANES)] = (lax.iota(jnp.int32, LANES) + g) * (n + 1)
        diag_cp = pltpu.async_copy(kflat.at[didx_v], diag_v, dsem)

        def chunk_base(c):
            cid = wid * NCHUNK + c
            return jnp.minimum(cid * chunk, t - chunk)

        def stage_fire(c):
            b = c % DEPTH
            base = chunk_base(c)
            scps = (pltpu.async_copy(ti.at[pl.ds(base, chunk)], iv_b[b], ssem),
                    pltpu.async_copy(tj.at[pl.ds(base, chunk)], jv_b[b], ssem),
                    pltpu.async_copy(tl.at[pl.ds(base, chunk)], lv_b[b], ssem))
            for cp in scps:
                cp.wait()
            iv_v, jv_v, lv_v = iv_b[b], jv_b[b], lv_b[b]
            il, ij = idx_il[b], idx_ij[b]

            @plsc.parallel_loop(0, chunk, step=LANES, unroll=UNROLL)
            def _(g):
                iv = iv_v[pl.ds(g, LANES)]
                row = iv * n
                il[pl.ds(g, LANES)] = row + lv_v[pl.ds(g, LANES)]
                ij[pl.ds(g, LANES)] = row + jv_v[pl.ds(g, LANES)]

            cps = (pltpu.async_copy(kflat.at[il], val_il[b], sems[b]),
                   pltpu.async_copy(kflat.at[ij], val_ij[b], sems[b]))
            return cps

        def drain_compute(c, cps, acc):
            b = c % DEPTH
            cps[0].wait()
            cps[1].wait()
            iv_v, jv_v, lv_v = iv_b[b], jv_b[b], lv_b[b]
            vil_v, vij_v = val_il[b], val_ij[b]
            cid = wid * NCHUNK + c
            thresh = cid * chunk - chunk_base(c)

            @plsc.parallel_loop(0, chunk, step=LANES, unroll=UNROLL,
                                carry=acc)
            def acc_loop(g, a):
                iv = iv_v[pl.ds(g, LANES)]
                jv = jv_v[pl.ds(g, LANES)]
                lv = lv_v[pl.ds(g, LANES)]
                di = plsc.load_gather(diag_v, [iv])
                dj = plsc.load_gather(diag_v, [jv])
                dl = plsc.load_gather(diag_v, [lv])
                vil = vil_v[pl.ds(g, LANES)]
                vij = vij_v[pl.ds(g, LANES)]
                d_il = (-2.0 * vil + di) + dl
                d_ij = (-2.0 * vij + di) + dj
                numer = jnp.maximum(MU + d_il, EPS)
                denom = jnp.maximum((2.0 * MU + d_ij) + d_il, EPS)
                term = _vln(denom / numer)
                keep = (lax.iota(jnp.int32, LANES) + g) >= thresh
                return a + jnp.where(keep, term, 0.0)

            return acc_loop

        acc = jnp.zeros((LANES,), jnp.float32)
        inflight = [stage_fire(c) for c in range(DEPTH - 1)]
        diag_cp.wait()
        for c in range(NCHUNK):
            if c + DEPTH - 1 < NCHUNK:
                inflight.append(stage_fire(c + DEPTH - 1))
            acc = drain_compute(c, inflight.pop(0), acc)

        part_v[...] = acc
        pltpu.sync_copy(part_v, out.at[pl.ds(wid * LANES, LANES)])

    return sc_kernel


def kernel(k, triplets):
    n = k.shape[0]
    t = triplets.shape[0]
    chunk = -(-t // TOTAL_CHUNKS)
    chunk = -(-chunk // LANES) * LANES

    kflat = k.reshape(-1)
    ti = triplets[:, 0]
    tj = triplets[:, 1]
    tl = triplets[:, 2]

    partials = _sc_loss_partials(n, t, chunk)(kflat, ti, tj, tl)
    return jnp.sum(partials)
